# baseline (speedup 1.0000x reference)
import jax
import jax.numpy as jnp
from jax import lax
from jax.experimental import pallas as pl
from jax.experimental.pallas import tpu as pltpu
from jax.experimental.pallas import tpu_sc as plsc

HIDDEN = 768
NSL = HIDDEN // 16
NPR = HIDDEN // 32
EPS = 1e-12
T = 16
HT = 8
K_BUF = 4


def _rsqrt16(x):
    i = plsc.bitcast(x, jnp.int32)
    i = jnp.int32(0x5F3759DF) - (i >> 1)
    y = plsc.bitcast(i, jnp.float32)
    for _ in range(4):
        y = y * (1.5 - 0.5 * x * y * y)
    return y


def _sc_body(idall_h, wemb_h, pemb_h, gam_h, bet_h, out_h,
             ids_v, w0, w1, w2, w3, p0, p1, p2, p3, g_v, b_v,
             s1_v, s2_v, sg0, sg1, sg2, sg3, so0, so1, so2, so3):
    ws = (w0, w1, w2, w3)
    ps = (p0, p1, p2, p3)
    sg = (sg0, sg1, sg2, sg3)
    so = (so0, so1, so2, so3)

    info = plsc.get_sparse_core_info()
    nw = info.num_cores * info.num_subcores
    wid = lax.axis_index("s") * info.num_cores + lax.axis_index("c")
    total = idall_h.shape[1]
    per_w = total // nw
    n_ch = per_w // T
    base = wid * per_w
    inv_h = 1.0 / HIDDEN
    himask = jnp.int32(-65536)

    pltpu.sync_copy(idall_h.at[:, pl.ds(base, per_w)], ids_v)
    pltpu.sync_copy(gam_h, g_v)
    pltpu.sync_copy(bet_h, b_v)

    def issue_gathers(c, b):
        offl = c * T
        pltpu.async_copy(wemb_h.at[ids_v.at[0, pl.ds(offl, T)]], ws[b], sg[b])
        pltpu.async_copy(pemb_h.at[ids_v.at[1, pl.ds(offl, T)]], ps[b], sg[b])

    for b in range(2):
        issue_gathers(b, b)

    def slot(c, b):
        offl = c * T
        off = base + offl
        pltpu.make_async_copy(
            wemb_h.at[ids_v.at[0, pl.ds(offl, T)]], ws[b], sg[b]).wait()
        pltpu.make_async_copy(
            pemb_h.at[ids_v.at[1, pl.ds(offl, T)]], ps[b], sg[b]).wait()

        w_v = ws[b]
        p_v = ps[b]
        zeros = jnp.zeros((16,), jnp.float32)

        for h in range(T // HT):
            t0 = h * HT

            def pass1(j, carry):
                s1 = list(carry[:HT])
                s2 = list(carry[HT:])
                sl0 = pl.ds(j * 32, 16)
                sl1 = pl.ds(j * 32 + 16, 16)
                slp = pl.ds(j * 16, 16)
                for t in range(t0, t0 + HT):
                    pv = p_v[t, slp]
                    pa = plsc.bitcast(pv << 16, jnp.float32)
                    pb = plsc.bitcast(pv & himask, jnp.float32)
                    e0 = w_v[t, sl0] + pa
                    e1 = w_v[t, sl1] + pb
                    w_v[t, sl0] = e0
                    w_v[t, sl1] = e1
                    i = t - t0
                    s1[i] = s1[i] + (e0 + e1)
                    s2[i] = s2[i] + e0 * e0 + e1 * e1
                return tuple(s1) + tuple(s2)

            carry = lax.fori_loop(0, NPR, pass1, (zeros,) * (2 * HT))
            for i in range(HT):
                s1_v[t0 + i, pl.ds(0, 16)] = carry[i]
                s2_v[t0 + i, pl.ds(0, 16)] = carry[HT + i]

        rows = jnp.arange(T, dtype=jnp.int32)
        m = zeros
        q = zeros
        for l in range(16):
            li = jnp.full((16,), l, jnp.int32)
            m = m + plsc.load_gather(s1_v, [rows, li])
            q = q + plsc.load_gather(s2_v, [rows, li])
        muv = m * inv_h
        varv = q * inv_h - muv * muv + EPS
        rv = _rsqrt16(varv)
        mu = [muv[t] for t in range(T)]
        rs = [rv[t] for t in range(T)]

        def pass2(j, _):
            sl0 = pl.ds(j * 32, 16)
            sl1 = pl.ds(j * 32 + 16, 16)
            ga = g_v[sl0]
            gb = g_v[sl1]
            ba = b_v[sl0]
            bb = b_v[sl1]
            for t in range(T):
                a0 = ga * rs[t]
                a1 = gb * rs[t]
                e0 = w_v[t, sl0]
                e1 = w_v[t, sl1]
                w_v[t, sl0] = (e0 - mu[t]) * a0 + ba
                w_v[t, sl1] = (e1 - mu[t]) * a1 + bb
            return 0

        lax.fori_loop(0, NPR, pass2, 0)

        pltpu.async_copy(w_v, out_h.at[pl.ds(off, T)], so[b])

        n = c + 2
        bn = (b + 2) % K_BUF

        @pl.when(jnp.logical_and(n >= K_BUF, n < n_ch))
        def _():
            pltpu.make_async_copy(
                ws[bn], out_h.at[pl.ds(base + (n - K_BUF) * T, T)],
                so[bn]).wait()

        @pl.when(n < n_ch)
        def _():
            issue_gathers(n, bn)

        return 0

    def group(gi, _):
        for b in range(K_BUF):
            slot(gi * K_BUF + b, b)
        return 0

    lax.fori_loop(0, n_ch // K_BUF, group, 0)

    for b in range(K_BUF):
        pltpu.make_async_copy(
            ws[b], out_h.at[pl.ds(base + (n_ch - K_BUF + b) * T, T)],
            so[b]).wait()


@jax.jit
def _run(idall, word_emb, pt_i32, ln_gamma, ln_beta):
    total = idall.shape[1]
    mesh = plsc.VectorSubcoreMesh(core_axis_name="c", subcore_axis_name="s")
    info = plsc.get_sparse_core_info()
    per_w = total // (info.num_cores * info.num_subcores)
    k = pl.kernel(
        _sc_body,
        out_type=jax.ShapeDtypeStruct((total, HIDDEN), jnp.float32),
        mesh=mesh,
        compiler_params=pltpu.CompilerParams(needs_layout_passes=False),
        scratch_types=[
            pltpu.VMEM((2, per_w), jnp.int32),
        ] + [pltpu.VMEM((T, HIDDEN), jnp.float32)] * K_BUF
          + [pltpu.VMEM((T, HIDDEN // 2), jnp.int32)] * K_BUF + [
            pltpu.VMEM((HIDDEN,), jnp.float32),
            pltpu.VMEM((HIDDEN,), jnp.float32),
            pltpu.VMEM((T, 16), jnp.float32),
            pltpu.VMEM((T, 16), jnp.float32),
        ] + [pltpu.SemaphoreType.DMA] * 8,
    )
    return k(idall, word_emb, pt_i32, ln_gamma, ln_beta)


def kernel(input_ids, token_type_ids, position_ids, word_emb, pos_emb,
           type_emb, ln_gamma, ln_beta):
    bsz, seq = input_ids.shape
    idall = jnp.stack([
        input_ids.reshape(-1),
        position_ids.reshape(-1) * 2 + token_type_ids.reshape(-1),
    ])
    npos = pos_emb.shape[0]
    ntype = type_emb.shape[0]
    ptab = (pos_emb[:, None, :] + type_emb[None, :, :]).reshape(
        npos * ntype, HIDDEN)
    ptb = ptab.reshape(npos * ntype, NPR, 2, 16).astype(jnp.bfloat16)
    pt_i32 = lax.bitcast_convert_type(
        ptb.transpose(0, 1, 3, 2), jnp.int32).reshape(npos * ntype,
                                                      HIDDEN // 2)
    out = _run(idall, word_emb, pt_i32, ln_gamma, ln_beta)
    return out.reshape(bsz, seq, HIDDEN)


# speedup vs baseline: 1.1561x; 1.1561x over previous
import jax
import jax.numpy as jnp
from jax import lax
from jax.experimental import pallas as pl
from jax.experimental.pallas import tpu as pltpu
from jax.experimental.pallas import tpu_sc as plsc

HIDDEN = 768
NSL = HIDDEN // 16
NPR = HIDDEN // 32
EPS = 1e-12
T = 16
HT = 8
K_BUF = 4


def _rsqrt16(x):
    i = plsc.bitcast(x, jnp.int32)
    i = jnp.int32(0x5F3759DF) - (i >> 1)
    y = plsc.bitcast(i, jnp.float32)
    for _ in range(4):
        y = y * (1.5 - 0.5 * x * y * y)
    return y


def _sc_body(idall_h, wemb_h, pemb_h, gam_h, bet_h, out_h,
             ids_v, w0, w1, w2, w3, p0, p1, p2, p3, g_v, b_v,
             s1_v, s2_v, sg0, sg1, sg2, sg3, so0, so1, so2, so3):
    ws = (w0, w1, w2, w3)
    ps = (p0, p1, p2, p3)
    sg = (sg0, sg1, sg2, sg3)
    so = (so0, so1, so2, so3)

    info = plsc.get_sparse_core_info()
    nw = info.num_cores * info.num_subcores
    wid = lax.axis_index("s") * info.num_cores + lax.axis_index("c")
    total = idall_h.shape[1]
    per_w = total // nw
    n_ch = per_w // T
    base = wid * per_w
    inv_h = 1.0 / HIDDEN
    himask = jnp.int32(-65536)

    pltpu.sync_copy(idall_h.at[:, pl.ds(base, per_w)], ids_v)
    pltpu.sync_copy(gam_h, g_v)
    pltpu.sync_copy(bet_h, b_v)

    def issue_gathers(c, b):
        offl = c * T
        pltpu.async_copy(wemb_h.at[ids_v.at[0, pl.ds(offl, T)]], ws[b], sg[b])
        pltpu.async_copy(pemb_h.at[ids_v.at[1, pl.ds(offl, T)]], ps[b], sg[b])

    for b in range(2):
        issue_gathers(b, b)

    def slot(c, b):
        offl = c * T
        off = base + offl
        pltpu.make_async_copy(
            wemb_h.at[ids_v.at[0, pl.ds(offl, T)]], ws[b], sg[b]).wait()
        pltpu.make_async_copy(
            pemb_h.at[ids_v.at[1, pl.ds(offl, T)]], ps[b], sg[b]).wait()

        w_v = ws[b]
        p_v = ps[b]
        zeros = jnp.zeros((16,), jnp.float32)

        for h in range(T // HT):
            t0 = h * HT

            def pass1(j, carry):
                s1 = list(carry[:HT])
                s2 = list(carry[HT:])
                sl0 = pl.ds(j * 32, 16)
                sl1 = pl.ds(j * 32 + 16, 16)
                slp = pl.ds(j * 16, 16)
                for t in range(t0, t0 + HT):
                    pv = p_v[t, slp]
                    pa = plsc.bitcast(pv << 16, jnp.float32)
                    pb = plsc.bitcast(pv & himask, jnp.float32)
                    e0 = w_v[t, sl0] + pa
                    e1 = w_v[t, sl1] + pb
                    w_v[t, sl0] = e0
                    w_v[t, sl1] = e1
                    i = t - t0
                    s1[i] = s1[i] + (e0 + e1)
                    s2[i] = s2[i] + e0 * e0 + e1 * e1
                return tuple(s1) + tuple(s2)

            carry = lax.fori_loop(0, NPR, pass1, (zeros,) * (2 * HT))
            for i in range(HT):
                s1_v[t0 + i, pl.ds(0, 16)] = carry[i]
                s2_v[t0 + i, pl.ds(0, 16)] = carry[HT + i]

        rows = jnp.arange(T, dtype=jnp.int32)
        m = zeros
        q = zeros
        for l in range(16):
            li = jnp.full((16,), l, jnp.int32)
            m = m + plsc.load_gather(s1_v, [rows, li])
            q = q + plsc.load_gather(s2_v, [rows, li])
        muv = m * inv_h
        varv = q * inv_h - muv * muv + EPS
        rv = _rsqrt16(varv)
        mu = [muv[t] for t in range(T)]
        rs = [rv[t] for t in range(T)]

        def pass2(j, _):
            sl0 = pl.ds(j * 32, 16)
            sl1 = pl.ds(j * 32 + 16, 16)
            ga = g_v[sl0]
            gb = g_v[sl1]
            ba = b_v[sl0]
            bb = b_v[sl1]
            for t in range(T):
                a0 = ga * rs[t]
                a1 = gb * rs[t]
                e0 = w_v[t, sl0]
                e1 = w_v[t, sl1]
                w_v[t, sl0] = (e0 - mu[t]) * a0 + ba
                w_v[t, sl1] = (e1 - mu[t]) * a1 + bb
            return 0

        lax.fori_loop(0, NPR, pass2, 0)

        pltpu.async_copy(w_v, out_h.at[pl.ds(off, T)], so[b])

        n = c + 2
        bn = (b + 2) % K_BUF

        @pl.when(jnp.logical_and(n >= K_BUF, n < n_ch))
        def _():
            pltpu.make_async_copy(
                ws[bn], out_h.at[pl.ds(base + (n - K_BUF) * T, T)],
                so[bn]).wait()

        @pl.when(n < n_ch)
        def _():
            issue_gathers(n, bn)

        return 0

    def group(gi, _):
        for b in range(K_BUF):
            slot(gi * K_BUF + b, b)
        return 0

    lax.fori_loop(0, n_ch // K_BUF, group, 0)

    for b in range(K_BUF):
        pltpu.make_async_copy(
            ws[b], out_h.at[pl.ds(base + (n_ch - K_BUF + b) * T, T)],
            so[b]).wait()


@jax.jit
def _run(idall, word_emb, pt_i32, ln_gamma, ln_beta):
    total = idall.shape[1]
    mesh = plsc.VectorSubcoreMesh(core_axis_name="c", subcore_axis_name="s")
    info = plsc.get_sparse_core_info()
    per_w = total // (info.num_cores * info.num_subcores)
    k = pl.kernel(
        _sc_body,
        out_type=jax.ShapeDtypeStruct((total, HIDDEN), jnp.float32),
        mesh=mesh,
        compiler_params=pltpu.CompilerParams(needs_layout_passes=False),
        scratch_types=[
            pltpu.VMEM((2, per_w), jnp.int32),
        ] + [pltpu.VMEM((T, HIDDEN), jnp.float32)] * K_BUF
          + [pltpu.VMEM((T, HIDDEN // 2), jnp.int32)] * K_BUF + [
            pltpu.VMEM((HIDDEN,), jnp.float32),
            pltpu.VMEM((HIDDEN,), jnp.float32),
            pltpu.VMEM((T, 16), jnp.float32),
            pltpu.VMEM((T, 16), jnp.float32),
        ] + [pltpu.SemaphoreType.DMA] * 8,
    )
    return k(idall, word_emb, pt_i32, ln_gamma, ln_beta)


def kernel(input_ids, token_type_ids, position_ids, word_emb, pos_emb,
           type_emb, ln_gamma, ln_beta):
    bsz, seq = input_ids.shape
    idall = jnp.stack([
        input_ids.reshape(-1),
        position_ids.reshape(-1) * 2 + token_type_ids.reshape(-1),
    ])
    npos = pos_emb.shape[0]
    ntype = type_emb.shape[0]
    ptab = (pos_emb[:, None, :] + type_emb[None, :, :]).reshape(
        npos * ntype, HIDDEN)
    ptb = ptab.reshape(npos * ntype, NPR, 2, 16).astype(jnp.bfloat16)
    lo = lax.bitcast_convert_type(ptb[:, :, 0, :], jnp.uint16)
    hi = lax.bitcast_convert_type(ptb[:, :, 1, :], jnp.uint16)
    words = lo.astype(jnp.uint32) | (hi.astype(jnp.uint32) << 16)
    pt_i32 = lax.bitcast_convert_type(words, jnp.int32).reshape(
        npos * ntype, HIDDEN // 2)
    out = _run(idall, word_emb, pt_i32, ln_gamma, ln_beta)
    return out.reshape(bsz, seq, HIDDEN)


# speedup vs baseline: 1.5683x; 1.3565x over previous
import jax
import jax.numpy as jnp
from jax import lax
from jax.experimental import pallas as pl
from jax.experimental.pallas import tpu as pltpu
from jax.experimental.pallas import tpu_sc as plsc

HIDDEN = 768
NSL = HIDDEN // 16
NPR = HIDDEN // 32
EPS = 1e-12
T = 16
HT = 8
K_BUF = 4


def _rsqrt16(x):
    i = plsc.bitcast(x, jnp.int32)
    i = jnp.int32(0x5F3759DF) - (i >> 1)
    y = plsc.bitcast(i, jnp.float32)
    for _ in range(4):
        y = y * (1.5 - 0.5 * x * y * y)
    return y


def _sc_body(idall_h, wemb_h, pemb_h, gam_h, bet_h, out_h,
             ids_v, w0, w1, w2, w3, p0, p1, p2, p3, g_v, b_v,
             s1_v, s2_v, sg0, sg1, sg2, sg3, so0, so1, so2, so3):
    ws = (w0, w1, w2, w3)
    ps = (p0, p1, p2, p3)
    sg = (sg0, sg1, sg2, sg3)
    so = (so0, so1, so2, so3)

    info = plsc.get_sparse_core_info()
    nw = info.num_cores * info.num_subcores
    wid = lax.axis_index("s") * info.num_cores + lax.axis_index("c")
    total = idall_h.shape[1]
    per_w = total // nw
    n_ch = per_w // T
    base = wid * per_w
    inv_h = 1.0 / HIDDEN
    himask = jnp.int32(-65536)

    pltpu.sync_copy(idall_h.at[:, pl.ds(base, per_w)], ids_v)
    pltpu.sync_copy(gam_h, g_v)
    pltpu.sync_copy(bet_h, b_v)

    def issue_gathers(c, b):
        offl = c * T
        pltpu.async_copy(wemb_h.at[ids_v.at[0, pl.ds(offl, T)]], ws[b], sg[b])
        pltpu.async_copy(pemb_h.at[ids_v.at[1, pl.ds(offl, T)]], ps[b], sg[b])

    for b in range(2):
        issue_gathers(b, b)

    def slot(c, b):
        offl = c * T
        off = base + offl
        pltpu.make_async_copy(
            wemb_h.at[ids_v.at[0, pl.ds(offl, T)]], ws[b], sg[b]).wait()
        pltpu.make_async_copy(
            pemb_h.at[ids_v.at[1, pl.ds(offl, T)]], ps[b], sg[b]).wait()

        w_v = ws[b]
        p_v = ps[b]
        zeros = jnp.zeros((16,), jnp.float32)

        for h in range(T // HT):
            t0 = h * HT

            @plsc.parallel_loop(0, NPR, carry=(zeros,) * (2 * HT), unroll=2)
            def pass1(j, carry, t0=t0, w_v=w_v, p_v=p_v):
                s1 = list(carry[:HT])
                s2 = list(carry[HT:])
                sl0 = pl.ds(j * 32, 16)
                sl1 = pl.ds(j * 32 + 16, 16)
                slp = pl.ds(j * 16, 16)
                for t in range(t0, t0 + HT):
                    pv = p_v[t, slp]
                    pa = plsc.bitcast(pv << 16, jnp.float32)
                    pb = plsc.bitcast(pv & himask, jnp.float32)
                    e0 = w_v[t, sl0] + pa
                    e1 = w_v[t, sl1] + pb
                    w_v[t, sl0] = e0
                    w_v[t, sl1] = e1
                    i = t - t0
                    s1[i] = s1[i] + (e0 + e1)
                    s2[i] = s2[i] + e0 * e0 + e1 * e1
                return tuple(s1) + tuple(s2)

            carry = pass1
            for i in range(HT):
                s1_v[t0 + i, pl.ds(0, 16)] = carry[i]
                s2_v[t0 + i, pl.ds(0, 16)] = carry[HT + i]

        rows = jnp.arange(T, dtype=jnp.int32)
        m = zeros
        q = zeros
        for l in range(16):
            li = jnp.full((16,), l, jnp.int32)
            m = m + plsc.load_gather(s1_v, [rows, li])
            q = q + plsc.load_gather(s2_v, [rows, li])
        muv = m * inv_h
        varv = q * inv_h - muv * muv + EPS
        rv = _rsqrt16(varv)
        mu = [muv[t] for t in range(T)]
        rs = [rv[t] for t in range(T)]

        @plsc.parallel_loop(0, NPR, unroll=2)
        def pass2(j, w_v=w_v):
            sl0 = pl.ds(j * 32, 16)
            sl1 = pl.ds(j * 32 + 16, 16)
            ga = g_v[sl0]
            gb = g_v[sl1]
            ba = b_v[sl0]
            bb = b_v[sl1]
            for t in range(T):
                a0 = ga * rs[t]
                a1 = gb * rs[t]
                e0 = w_v[t, sl0]
                e1 = w_v[t, sl1]
                w_v[t, sl0] = (e0 - mu[t]) * a0 + ba
                w_v[t, sl1] = (e1 - mu[t]) * a1 + bb

        pltpu.async_copy(w_v, out_h.at[pl.ds(off, T)], so[b])

        n = c + 2
        bn = (b + 2) % K_BUF

        @pl.when(jnp.logical_and(n >= K_BUF, n < n_ch))
        def _():
            pltpu.make_async_copy(
                ws[bn], out_h.at[pl.ds(base + (n - K_BUF) * T, T)],
                so[bn]).wait()

        @pl.when(n < n_ch)
        def _():
            issue_gathers(n, bn)

        return 0

    def group(gi, _):
        for b in range(K_BUF):
            slot(gi * K_BUF + b, b)
        return 0

    lax.fori_loop(0, n_ch // K_BUF, group, 0)

    for b in range(K_BUF):
        pltpu.make_async_copy(
            ws[b], out_h.at[pl.ds(base + (n_ch - K_BUF + b) * T, T)],
            so[b]).wait()


@jax.jit
def _run(idall, word_emb, pt_i32, ln_gamma, ln_beta):
    total = idall.shape[1]
    mesh = plsc.VectorSubcoreMesh(core_axis_name="c", subcore_axis_name="s")
    info = plsc.get_sparse_core_info()
    per_w = total // (info.num_cores * info.num_subcores)
    k = pl.kernel(
        _sc_body,
        out_type=jax.ShapeDtypeStruct((total, HIDDEN), jnp.float32),
        mesh=mesh,
        compiler_params=pltpu.CompilerParams(needs_layout_passes=False),
        scratch_types=[
            pltpu.VMEM((2, per_w), jnp.int32),
        ] + [pltpu.VMEM((T, HIDDEN), jnp.float32)] * K_BUF
          + [pltpu.VMEM((T, HIDDEN // 2), jnp.int32)] * K_BUF + [
            pltpu.VMEM((HIDDEN,), jnp.float32),
            pltpu.VMEM((HIDDEN,), jnp.float32),
            pltpu.VMEM((T, 16), jnp.float32),
            pltpu.VMEM((T, 16), jnp.float32),
        ] + [pltpu.SemaphoreType.DMA] * 8,
    )
    return k(idall, word_emb, pt_i32, ln_gamma, ln_beta)


def kernel(input_ids, token_type_ids, position_ids, word_emb, pos_emb,
           type_emb, ln_gamma, ln_beta):
    bsz, seq = input_ids.shape
    idall = jnp.stack([
        input_ids.reshape(-1),
        position_ids.reshape(-1) * 2 + token_type_ids.reshape(-1),
    ])
    npos = pos_emb.shape[0]
    ntype = type_emb.shape[0]
    ptab = (pos_emb[:, None, :] + type_emb[None, :, :]).reshape(
        npos * ntype, HIDDEN)
    ptb = ptab.reshape(npos * ntype, NPR, 2, 16).astype(jnp.bfloat16)
    lo = lax.bitcast_convert_type(ptb[:, :, 0, :], jnp.uint16)
    hi = lax.bitcast_convert_type(ptb[:, :, 1, :], jnp.uint16)
    words = lo.astype(jnp.uint32) | (hi.astype(jnp.uint32) << 16)
    pt_i32 = lax.bitcast_convert_type(words, jnp.int32).reshape(
        npos * ntype, HIDDEN // 2)
    out = _run(idall, word_emb, pt_i32, ln_gamma, ln_beta)
    return out.reshape(bsz, seq, HIDDEN)
